# Initial kernel scaffold; baseline (speedup 1.0000x reference)
import jax
import jax.numpy as jnp
from jax.experimental import pallas as pl


def kernel(token_ids, weight):
    raise NotImplementedError("write your pallas kernel here")


# speedup vs baseline: 1.6841x; 1.6841x over previous
import functools

import jax
import jax.numpy as jnp
from jax import lax
from jax.experimental import pallas as pl
from jax.experimental.pallas import tpu as pltpu
from jax.experimental.pallas import tpu_sc as plsc

EMBEDDING_DIM = 64
R = 128


@functools.lru_cache(maxsize=None)
def _build(B, D, NC, NS):
    NW = NC * NS
    b_per_w = B // NW
    S = b_per_w // R

    mesh = plsc.VectorSubcoreMesh(core_axis_name="c", subcore_axis_name="s")

    @functools.partial(
        pl.kernel,
        mesh=mesh,
        out_type=jax.ShapeDtypeStruct((B, D), jnp.float32),
        scratch_types=[
            pltpu.VMEM((S, R), jnp.int32),
            pltpu.VMEM((R, D), jnp.float32),
            pltpu.SemaphoreType.DMA,
        ],
        compiler_params=pltpu.CompilerParams(use_tc_tiling_on_sc=False),
    )
    def gather_kernel(table_hbm, idx_hbm, out_hbm, idx_v, rows_v, sem):
        wid = lax.axis_index("s") * NC + lax.axis_index("c")
        base = wid * b_per_w
        pltpu.sync_copy(idx_hbm.at[wid], idx_v)

        def step(s, carry):
            pltpu.async_copy(table_hbm.at[idx_v.at[s]], rows_v, sem).wait()
            pltpu.sync_copy(rows_v, out_hbm.at[pl.ds(base + s * R, R)])
            return carry

        lax.fori_loop(0, S, step, 0)

    return gather_kernel


def kernel(token_ids, weight):
    B = token_ids.shape[0] * token_ids.shape[1]
    D = weight.shape[1]
    info = plsc.get_sparse_core_info()
    NC, NS = info.num_cores, info.num_subcores
    idx = token_ids.reshape(-1).astype(jnp.int32)
    idx3 = idx.reshape(NC * NS, B // (NC * NS) // R, R)
    out = _build(B, D, NC, NS)(weight, idx3)
    return out.reshape(token_ids.shape[0], token_ids.shape[1], D)


# speedup vs baseline: 1.8586x; 1.1036x over previous
import functools

import jax
import jax.numpy as jnp
from jax import lax
from jax.experimental import pallas as pl
from jax.experimental.pallas import tpu as pltpu
from jax.experimental.pallas import tpu_sc as plsc

EMBEDDING_DIM = 64
R = 128
G = 4


@functools.lru_cache(maxsize=None)
def _build(B, D, NC, NS):
    NW = NC * NS
    b_per_w = B // NW
    S = b_per_w // R
    T = S // (2 * G)

    mesh = plsc.VectorSubcoreMesh(core_axis_name="c", subcore_axis_name="s")

    @functools.partial(
        pl.kernel,
        mesh=mesh,
        out_type=jax.ShapeDtypeStruct((B, D), jnp.float32),
        scratch_types=[
            pltpu.VMEM((S, R), jnp.int32),
            pltpu.VMEM((G, R, D), jnp.float32),
            pltpu.VMEM((G, R, D), jnp.float32),
            pltpu.SemaphoreType.DMA,
            pltpu.SemaphoreType.DMA,
            pltpu.SemaphoreType.DMA,
            pltpu.SemaphoreType.DMA,
        ],
        compiler_params=pltpu.CompilerParams(use_tc_tiling_on_sc=False),
    )
    def gather_kernel(table_hbm, idx_hbm, out_hbm,
                      idx_v, rows_a, rows_b, gs_a, gs_b, os_a, os_b):
        wid = lax.axis_index("s") * NC + lax.axis_index("c")
        base = wid * b_per_w
        pltpu.sync_copy(idx_hbm.at[wid], idx_v)

        def fire_gathers(g, rows, sem):
            for b in range(G):
                s = jnp.minimum(g * G + b, S - 1)
                pltpu.async_copy(table_hbm.at[idx_v.at[s]], rows.at[b], sem)

        def wait_gathers(g, rows, sem):
            for b in range(G):
                s = jnp.minimum(g * G + b, S - 1)
                pltpu.make_async_copy(table_hbm.at[idx_v.at[s]], rows.at[b], sem).wait()

        def fire_outs(g, rows, sem):
            for b in range(G):
                s = g * G + b
                pltpu.async_copy(rows.at[b], out_hbm.at[pl.ds(base + s * R, R)], sem)

        def wait_outs(g, rows, sem):
            for b in range(G):
                s = g * G + b
                pltpu.make_async_copy(rows.at[b], out_hbm.at[pl.ds(base + s * R, R)], sem).wait()

        fire_gathers(0, rows_a, gs_a)
        fire_gathers(1, rows_b, gs_b)

        def body(t, carry):
            g0 = 2 * t
            wait_gathers(g0, rows_a, gs_a)
            fire_outs(g0, rows_a, os_a)
            wait_gathers(g0 + 1, rows_b, gs_b)
            fire_outs(g0 + 1, rows_b, os_b)
            wait_outs(g0, rows_a, os_a)
            fire_gathers(g0 + 2, rows_a, gs_a)
            wait_outs(g0 + 1, rows_b, os_b)
            fire_gathers(g0 + 3, rows_b, gs_b)
            return carry

        lax.fori_loop(0, T, body, 0)
        wait_gathers(2 * T, rows_a, gs_a)
        wait_gathers(2 * T + 1, rows_b, gs_b)

    return gather_kernel


def kernel(token_ids, weight):
    B = token_ids.shape[0] * token_ids.shape[1]
    D = weight.shape[1]
    info = plsc.get_sparse_core_info()
    NC, NS = info.num_cores, info.num_subcores
    idx = token_ids.reshape(-1).astype(jnp.int32)
    idx3 = idx.reshape(NC * NS, B // (NC * NS) // R, R)
    out = _build(B, D, NC, NS)(weight, idx3)
    return out.reshape(token_ids.shape[0], token_ids.shape[1], D)
